# Initial kernel scaffold; baseline (speedup 1.0000x reference)
import jax
import jax.numpy as jnp
from jax.experimental import pallas as pl


def kernel(points, point_features, boxes3d):
    raise NotImplementedError("write your pallas kernel here")


# speedup vs baseline: 68.9399x; 68.9399x over previous
import functools

import jax
import jax.numpy as jnp
from jax import lax
from jax.experimental import pallas as pl
from jax.experimental.pallas import tpu as pltpu
from jax.experimental.pallas import tpu_sc as plsc

_B, _N, _C, _M, _S = 4, 16384, 128, 128, 512
_D = 3 + _C
_EXTRA = 1.0
_L = 16
_NW = 32
_BPW = (_B * _M) // _NW
_WPB = _M // _BPW
_NVEC = _N // _L
_NP1 = _N + 1
_DP = 144

_GDN = lax.GatherDimensionNumbers(
    offset_dims=(), collapsed_slice_dims=(0,), start_index_map=(0,))


def _vpermute(x, idx):
    return lax.gather(x, idx[:, None], _GDN, slice_sizes=(1,),
                      mode=lax.GatherScatterMode.PROMISE_IN_BOUNDS)


def _sc_body(xs, ys, zs, boxp, data, out, flags_out,
             xs_v, ys_v, zs_v, boxp_v, idx_buf, pidx_buf, rows_v, flags_v,
             gsem, osem):
    cid = lax.axis_index("c")
    sid = lax.axis_index("s")
    wid = sid * 2 + cid
    b = wid // _WPB
    mg = (wid % _WPB) * _BPW

    pltpu.sync_copy(xs.at[b], xs_v)
    pltpu.sync_copy(ys.at[b], ys_v)
    pltpu.sync_copy(zs.at[b], zs_v)

    lane = lax.iota(jnp.int32, _L)

    def box_body(k, flag_vec):
        m = mg + k
        pltpu.sync_copy(boxp.at[b, m], boxp_v)
        cx = boxp_v[0]
        cy = boxp_v[1]
        cz = boxp_v[2]
        hx = boxp_v[3]
        hy = boxp_v[4]
        hz = boxp_v[5]
        ca = boxp_v[6]
        sa = boxp_v[7]

        s_cap = jnp.full((_L,), _S, jnp.int32)
        ones_v = jnp.full((_L,), 1, jnp.int32)
        zeros_v = jnp.full((_L,), 0, jnp.int32)

        def scan_body(i, off_v):
            xv = xs_v[pl.ds(i * _L, _L)]
            yv = ys_v[pl.ds(i * _L, _L)]
            zv = zs_v[pl.ds(i * _L, _L)]
            px = xv - cx
            py = yv - cy
            pz = zv - cz
            lx = px * ca + py * sa
            ly = py * ca - px * sa
            mask = ((jnp.abs(lx) < hx) & (jnp.abs(ly) < hy)
                    & (jnp.abs(pz) <= hz))
            c = jnp.where(mask, ones_v, zeros_v)
            for d in (1, 2, 4, 8):
                dv = jnp.full((_L,), d, jnp.int32)
                sh = _vpermute(c, jnp.maximum(lane - dv, zeros_v))
                c = c + jnp.where(lane >= dv, sh, zeros_v)
            pos = c + off_v - ones_v
            ok = mask & (pos < s_cap)
            pos_safe = jnp.where(ok, pos, s_cap)
            ptid = lane + jnp.full((_L,), i * _L, jnp.int32)
            plsc.store_scatter(idx_buf, [pos_safe], ptid)
            return off_v + _vpermute(c, jnp.full((_L,), _L - 1, jnp.int32))

        cnt_all_v = lax.fori_loop(0, _NVEC, scan_body,
                                  jnp.zeros((_L,), jnp.int32))

        cnt_v = jnp.minimum(cnt_all_v, s_cap)
        cntc_v = jnp.maximum(cnt_v, ones_v)
        nonempty = cnt_v > zeros_v
        base_v = jnp.full((_L,), b * _NP1, jnp.int32)
        zrow_v = jnp.full((_L,), b * _NP1 + _N, jnp.int32)
        for j in range(_S // _L):
            sv = lane + jnp.full((_L,), j * _L, jnp.int32)
            rv = jnp.remainder(sv, cntc_v)
            g = plsc.load_gather(idx_buf, [rv])
            g = jnp.where(nonempty, g + base_v, zrow_v)
            pidx_buf[j // 8, pl.ds((j % 8) * _L, _L)] = g

        @pl.when(k > 0)
        def _():
            pltpu.make_async_copy(rows_v, out.at[b, m - 1], osem).wait()

        descs = [
            pltpu.async_copy(data.at[pidx_buf.at[j]],
                             rows_v.at[pl.ds(j * 128, 128)], gsem)
            for j in range(_S // 128)
        ]
        for d in descs:
            d.wait()

        pltpu.async_copy(rows_v, out.at[b, m], osem)

        flag = jnp.where(cnt_all_v == zeros_v, ones_v, zeros_v)
        k_v = jnp.full((_L,), k, jnp.int32)
        return jnp.where(lane == k_v, flag, flag_vec)

    flags = lax.fori_loop(0, _BPW, box_body, jnp.zeros((_L,), jnp.int32))
    pltpu.make_async_copy(rows_v, out.at[b, mg + _BPW - 1], osem).wait()
    flags_v[...] = flags
    pltpu.sync_copy(flags_v, flags_out.at[b, pl.ds(mg, _L)])


@functools.partial(
    pl.kernel,
    mesh=plsc.VectorSubcoreMesh(core_axis_name="c", subcore_axis_name="s"),
    compiler_params=pltpu.CompilerParams(needs_layout_passes=False,
                                         use_tc_tiling_on_sc=False),
    out_type=[
        jax.ShapeDtypeStruct((_B, _M, _S, _DP), jnp.float32),
        jax.ShapeDtypeStruct((_B, _M), jnp.int32),
    ],
    scratch_types=[
        pltpu.VMEM((_N,), jnp.float32),
        pltpu.VMEM((_N,), jnp.float32),
        pltpu.VMEM((_N,), jnp.float32),
        pltpu.VMEM((8, _L), jnp.float32),
        pltpu.VMEM((_S + _L,), jnp.int32),
        pltpu.VMEM((_S // 128, 128), jnp.int32),
        pltpu.VMEM((_S, _DP), jnp.float32),
        pltpu.VMEM((_L,), jnp.int32),
        pltpu.SemaphoreType.DMA,
        pltpu.SemaphoreType.DMA,
    ],
)
def _sc_pool(*args):
    _sc_body(*args)


def kernel(points, point_features, boxes3d):
    points = points.astype(jnp.float32)
    point_features = point_features.astype(jnp.float32)
    boxes3d = boxes3d.astype(jnp.float32)

    xs = points[..., 0]
    ys = points[..., 1]
    zs = points[..., 2]
    half = boxes3d[..., 3:6] * 0.5 + _EXTRA
    rz = boxes3d[..., 6]
    boxp = jnp.stack(
        [boxes3d[..., 0], boxes3d[..., 1], boxes3d[..., 2],
         half[..., 0], half[..., 1], half[..., 2],
         jnp.cos(rz), jnp.sin(rz)], axis=-1)
    boxp = (boxp[..., None] * jnp.ones((_L,), jnp.float32))

    data = jnp.concatenate([points, point_features], axis=-1)
    data = jnp.pad(data, ((0, 0), (0, 1), (0, _DP - _D)))
    data = data.reshape(_B * _NP1, _DP)

    pooled, flags = _sc_pool(xs, ys, zs, boxp, data)
    return pooled[..., :_D], flags


# speedup vs baseline: 85.9993x; 1.2475x over previous
import functools

import jax
import jax.numpy as jnp
from jax import lax
from jax.experimental import pallas as pl
from jax.experimental.pallas import tpu as pltpu
from jax.experimental.pallas import tpu_sc as plsc

_B, _N, _C, _M, _S = 4, 16384, 128, 128, 512
_D = 3 + _C
_EXTRA = 1.0
_L = 16
_NW = 32
_BPW = (_B * _M) // _NW
_WPB = _M // _BPW
_NVEC = _N // _L
_NP1 = _N + 1
_DP = 144

_GDN = lax.GatherDimensionNumbers(
    offset_dims=(), collapsed_slice_dims=(0,), start_index_map=(0,))


def _vpermute(x, idx):
    return lax.gather(x, idx[:, None], _GDN, slice_sizes=(1,),
                      mode=lax.GatherScatterMode.PROMISE_IN_BOUNDS)


def _sc_body(xs, ys, zs, boxp, data, out, flags_out,
             xs_v, ys_v, zs_v, boxp_v, idx_buf, pidx_buf, rows_v, flags_v,
             gsem, osem):
    cid = lax.axis_index("c")
    sid = lax.axis_index("s")
    wid = sid * 2 + cid
    b = wid // _WPB
    mg = (wid % _WPB) * _BPW

    pltpu.sync_copy(xs.at[b], xs_v)
    pltpu.sync_copy(ys.at[b], ys_v)
    pltpu.sync_copy(zs.at[b], zs_v)

    lane = lax.iota(jnp.int32, _L)

    def box_body(k, flag_vec):
        m = mg + k
        pltpu.sync_copy(boxp.at[b, m], boxp_v)
        cx = boxp_v[0]
        cy = boxp_v[1]
        cz = boxp_v[2]
        hx = boxp_v[3]
        hy = boxp_v[4]
        hz = boxp_v[5]
        ca = boxp_v[6]
        sa = boxp_v[7]

        s_cap = jnp.full((_L,), _S, jnp.int32)
        ones_v = jnp.full((_L,), 1, jnp.int32)
        zeros_v = jnp.full((_L,), 0, jnp.int32)

        def scan_cond(carry):
            i, off_v = carry
            return (i < _NVEC) & jnp.all(off_v < s_cap)

        def scan_body(carry):
            i, off_v = carry
            xv = xs_v[pl.ds(i * _L, _L)]
            yv = ys_v[pl.ds(i * _L, _L)]
            zv = zs_v[pl.ds(i * _L, _L)]
            px = xv - cx
            py = yv - cy
            pz = zv - cz
            lx = px * ca + py * sa
            ly = py * ca - px * sa
            mask = ((jnp.abs(lx) < hx) & (jnp.abs(ly) < hy)
                    & (jnp.abs(pz) <= hz))
            mi = jnp.where(mask, ones_v, zeros_v)
            pos = plsc.cumsum(mi) + off_v - ones_v
            ok = mask & (pos < s_cap)
            ptid = lane + jnp.full((_L,), i * _L, jnp.int32)
            plsc.store_scatter(idx_buf, [pos], ptid, mask=ok)
            return i + 1, off_v + plsc.all_reduce_population_count(mask)

        _, cnt_all_v = lax.while_loop(
            scan_cond, scan_body, (jnp.int32(0), jnp.zeros((_L,), jnp.int32)))

        cnt_v = jnp.minimum(cnt_all_v, s_cap)
        cntc_v = jnp.maximum(cnt_v, ones_v)
        nonempty = cnt_v > zeros_v
        base_v = jnp.full((_L,), b * _NP1, jnp.int32)
        zrow_v = jnp.full((_L,), b * _NP1 + _N, jnp.int32)
        for j in range(_S // _L):
            sv = lane + jnp.full((_L,), j * _L, jnp.int32)
            rv = jnp.remainder(sv, cntc_v)
            g = plsc.load_gather(idx_buf, [rv])
            g = jnp.where(nonempty, g + base_v, zrow_v)
            pidx_buf[j // 8, pl.ds((j % 8) * _L, _L)] = g

        @pl.when(k > 0)
        def _():
            pltpu.make_async_copy(rows_v, out.at[b, m - 1], osem).wait()

        descs = [
            pltpu.async_copy(data.at[pidx_buf.at[j]],
                             rows_v.at[pl.ds(j * 128, 128)], gsem)
            for j in range(_S // 128)
        ]
        for d in descs:
            d.wait()

        pltpu.async_copy(rows_v, out.at[b, m], osem)

        flag = jnp.where(cnt_all_v == zeros_v, ones_v, zeros_v)
        k_v = jnp.full((_L,), k, jnp.int32)
        return jnp.where(lane == k_v, flag, flag_vec)

    flags = lax.fori_loop(0, _BPW, box_body, jnp.zeros((_L,), jnp.int32))
    pltpu.make_async_copy(rows_v, out.at[b, mg + _BPW - 1], osem).wait()
    flags_v[...] = flags
    pltpu.sync_copy(flags_v, flags_out.at[b, pl.ds(mg, _L)])


@functools.partial(
    pl.kernel,
    mesh=plsc.VectorSubcoreMesh(core_axis_name="c", subcore_axis_name="s"),
    compiler_params=pltpu.CompilerParams(needs_layout_passes=False,
                                         use_tc_tiling_on_sc=False),
    out_type=[
        jax.ShapeDtypeStruct((_B, _M, _S, _DP), jnp.float32),
        jax.ShapeDtypeStruct((_B, _M), jnp.int32),
    ],
    scratch_types=[
        pltpu.VMEM((_N,), jnp.float32),
        pltpu.VMEM((_N,), jnp.float32),
        pltpu.VMEM((_N,), jnp.float32),
        pltpu.VMEM((8, _L), jnp.float32),
        pltpu.VMEM((_S + _L,), jnp.int32),
        pltpu.VMEM((_S // 128, 128), jnp.int32),
        pltpu.VMEM((_S, _DP), jnp.float32),
        pltpu.VMEM((_L,), jnp.int32),
        pltpu.SemaphoreType.DMA,
        pltpu.SemaphoreType.DMA,
    ],
)
def _sc_pool(*args):
    _sc_body(*args)


def kernel(points, point_features, boxes3d):
    points = points.astype(jnp.float32)
    point_features = point_features.astype(jnp.float32)
    boxes3d = boxes3d.astype(jnp.float32)

    xs = points[..., 0]
    ys = points[..., 1]
    zs = points[..., 2]
    half = boxes3d[..., 3:6] * 0.5 + _EXTRA
    rz = boxes3d[..., 6]
    boxp = jnp.stack(
        [boxes3d[..., 0], boxes3d[..., 1], boxes3d[..., 2],
         half[..., 0], half[..., 1], half[..., 2],
         jnp.cos(rz), jnp.sin(rz)], axis=-1)
    boxp = (boxp[..., None] * jnp.ones((_L,), jnp.float32))

    data = jnp.concatenate([points, point_features], axis=-1)
    data = jnp.pad(data, ((0, 0), (0, 1), (0, _DP - _D)))
    data = data.reshape(_B * _NP1, _DP)

    pooled, flags = _sc_pool(xs, ys, zs, boxp, data)
    return pooled[..., :_D], flags


# speedup vs baseline: 89.3075x; 1.0385x over previous
import functools

import jax
import jax.numpy as jnp
from jax import lax
from jax.experimental import pallas as pl
from jax.experimental.pallas import tpu as pltpu
from jax.experimental.pallas import tpu_sc as plsc

_B, _N, _C, _M, _S = 4, 16384, 128, 128, 512
_D = 3 + _C
_EXTRA = 1.0
_L = 16
_NW = 32
_BPW = (_B * _M) // _NW
_WPB = _M // _BPW
_NVEC = _N // _L
_CHUNK = 8
_NP1 = _N + 1
_DP = 144

_GDN = lax.GatherDimensionNumbers(
    offset_dims=(), collapsed_slice_dims=(0,), start_index_map=(0,))


def _vpermute(x, idx):
    return lax.gather(x, idx[:, None], _GDN, slice_sizes=(1,),
                      mode=lax.GatherScatterMode.PROMISE_IN_BOUNDS)


def _sc_body(xs, ys, zs, boxp, data, out, flags_out,
             xs_v, ys_v, zs_v, boxp_v, idx_buf, pidx_buf, rows_v, flags_v,
             gsem, osem):
    cid = lax.axis_index("c")
    sid = lax.axis_index("s")
    wid = sid * 2 + cid
    b = wid // _WPB
    mg = (wid % _WPB) * _BPW

    pltpu.sync_copy(xs.at[b], xs_v)
    pltpu.sync_copy(ys.at[b], ys_v)
    pltpu.sync_copy(zs.at[b], zs_v)

    lane = lax.iota(jnp.int32, _L)

    def box_body(k, flag_vec):
        m = mg + k
        pltpu.sync_copy(boxp.at[b, m], boxp_v)
        cx = boxp_v[0]
        cy = boxp_v[1]
        cz = boxp_v[2]
        hx = boxp_v[3]
        hy = boxp_v[4]
        hz = boxp_v[5]
        ca = boxp_v[6]
        sa = boxp_v[7]

        s_cap = jnp.full((_L,), _S, jnp.int32)
        ones_v = jnp.full((_L,), 1, jnp.int32)
        zeros_v = jnp.full((_L,), 0, jnp.int32)

        def scan_cond(carry):
            i, off_v = carry
            return (i < _NVEC // _CHUNK) & jnp.all(off_v < s_cap)

        def scan_body(carry):
            i, off_v = carry
            base = i * (_CHUNK * _L)
            for t in range(_CHUNK):
                xv = xs_v[pl.ds(base + t * _L, _L)]
                yv = ys_v[pl.ds(base + t * _L, _L)]
                zv = zs_v[pl.ds(base + t * _L, _L)]
                px = xv - cx
                py = yv - cy
                pz = zv - cz
                lx = px * ca + py * sa
                ly = py * ca - px * sa
                mask = ((jnp.abs(lx) < hx) & (jnp.abs(ly) < hy)
                        & (jnp.abs(pz) <= hz))
                mi = jnp.where(mask, ones_v, zeros_v)
                pos = plsc.cumsum(mi) + off_v - ones_v
                ok = mask & (pos < s_cap)
                ptid = lane + jnp.full((_L,), t * _L, jnp.int32) \
                    + jnp.full((_L,), base, jnp.int32)
                plsc.store_scatter(idx_buf, [pos], ptid, mask=ok)
                off_v = off_v + plsc.all_reduce_population_count(mask)
            return i + 1, off_v

        _, cnt_all_v = lax.while_loop(
            scan_cond, scan_body, (jnp.int32(0), jnp.zeros((_L,), jnp.int32)))

        cnt_v = jnp.minimum(cnt_all_v, s_cap)
        cntc_v = jnp.maximum(cnt_v, ones_v)
        nonempty = cnt_v > zeros_v
        base_v = jnp.full((_L,), b * _NP1, jnp.int32)
        zrow_v = jnp.full((_L,), b * _NP1 + _N, jnp.int32)
        for j in range(_S // _L):
            sv = lane + jnp.full((_L,), j * _L, jnp.int32)
            rv = jnp.remainder(sv, cntc_v)
            g = plsc.load_gather(idx_buf, [rv])
            g = jnp.where(nonempty, g + base_v, zrow_v)
            pidx_buf[j // 8, pl.ds((j % 8) * _L, _L)] = g

        @pl.when(k > 0)
        def _():
            pltpu.make_async_copy(rows_v, out.at[b, m - 1], osem).wait()

        descs = [
            pltpu.async_copy(data.at[pidx_buf.at[j]],
                             rows_v.at[pl.ds(j * 128, 128)], gsem)
            for j in range(_S // 128)
        ]
        for d in descs:
            d.wait()

        pltpu.async_copy(rows_v, out.at[b, m], osem)

        flag = jnp.where(cnt_all_v == zeros_v, ones_v, zeros_v)
        k_v = jnp.full((_L,), k, jnp.int32)
        return jnp.where(lane == k_v, flag, flag_vec)

    flags = lax.fori_loop(0, _BPW, box_body, jnp.zeros((_L,), jnp.int32))
    pltpu.make_async_copy(rows_v, out.at[b, mg + _BPW - 1], osem).wait()
    flags_v[...] = flags
    pltpu.sync_copy(flags_v, flags_out.at[b, pl.ds(mg, _L)])


@functools.partial(
    pl.kernel,
    mesh=plsc.VectorSubcoreMesh(core_axis_name="c", subcore_axis_name="s"),
    compiler_params=pltpu.CompilerParams(needs_layout_passes=False,
                                         use_tc_tiling_on_sc=False),
    out_type=[
        jax.ShapeDtypeStruct((_B, _M, _S, _DP), jnp.float32),
        jax.ShapeDtypeStruct((_B, _M), jnp.int32),
    ],
    scratch_types=[
        pltpu.VMEM((_N,), jnp.float32),
        pltpu.VMEM((_N,), jnp.float32),
        pltpu.VMEM((_N,), jnp.float32),
        pltpu.VMEM((8, _L), jnp.float32),
        pltpu.VMEM((_S + _L,), jnp.int32),
        pltpu.VMEM((_S // 128, 128), jnp.int32),
        pltpu.VMEM((_S, _DP), jnp.float32),
        pltpu.VMEM((_L,), jnp.int32),
        pltpu.SemaphoreType.DMA,
        pltpu.SemaphoreType.DMA,
    ],
)
def _sc_pool(*args):
    _sc_body(*args)


def kernel(points, point_features, boxes3d):
    points = points.astype(jnp.float32)
    point_features = point_features.astype(jnp.float32)
    boxes3d = boxes3d.astype(jnp.float32)

    xs = points[..., 0]
    ys = points[..., 1]
    zs = points[..., 2]
    half = boxes3d[..., 3:6] * 0.5 + _EXTRA
    rz = boxes3d[..., 6]
    boxp = jnp.stack(
        [boxes3d[..., 0], boxes3d[..., 1], boxes3d[..., 2],
         half[..., 0], half[..., 1], half[..., 2],
         jnp.cos(rz), jnp.sin(rz)], axis=-1)
    boxp = (boxp[..., None] * jnp.ones((_L,), jnp.float32))

    data = jnp.concatenate([points, point_features], axis=-1)
    data = jnp.pad(data, ((0, 0), (0, 1), (0, _DP - _D)))
    data = data.reshape(_B * _NP1, _DP)

    pooled, flags = _sc_pool(xs, ys, zs, boxp, data)
    return pooled[..., :_D], flags
